# baseline (speedup 1.0000x reference)
import functools

import jax
import jax.numpy as jnp
from jax import lax
from jax.experimental import pallas as pl
from jax.experimental.pallas import tpu as pltpu
from jax.experimental.pallas import tpu_sc as plsc

N = 10000
E = 320000
C = 128
N_P = 10240

NC = 2
NS = 16
NW = NC * NS
EPW = E // NW
CHUNK = 80
NCHUNK = EPW // CHUNK
RCHUNK = N_P // CHUNK
RPS = RCHUNK // NS

_mesh = plsc.VectorSubcoreMesh(
    core_axis_name="c", subcore_axis_name="s", num_cores=NC, num_subcores=NS
)


def _fill_vec16(ref, nwords, value):
    val = jnp.full((16,), value, dtype=jnp.float32)

    @pl.loop(0, nwords // 16)
    def _(i):
        ref[pl.ds(i * 16, 16)] = val


NPS = N_P // NS


@functools.partial(
    pl.kernel,
    out_type=jax.ShapeDtypeStruct((NC * N_P,), jnp.float32),
    mesh=_mesh,
    scratch_types=[
        pltpu.VMEM((NCHUNK, CHUNK), jnp.int32),
        pltpu.VMEM((CHUNK,), jnp.float32),
        pltpu.VMEM((NPS,), jnp.float32),
        pltpu.VMEM_SHARED((N_P,), jnp.float32),
        pltpu.SemaphoreType.DMA,
        pltpu.SemaphoreType.DMA,
    ],
)
def _deg_kernel(col3d_hbm, out_hbm, cidx_all, ones_v, tmp_v, deg_sh, semi, sems):
    cid = lax.axis_index("c")
    sid = lax.axis_index("s")
    wid = sid * NC + cid

    idx_load = pltpu.async_copy(col3d_hbm.at[wid], cidx_all, semi)

    _fill_vec16(ones_v, CHUNK, 1.0)
    _fill_vec16(tmp_v, NPS, 0.0)

    pltpu.sync_copy(tmp_v, deg_sh.at[pl.ds(sid * NPS, NPS)])
    idx_load.wait()
    plsc.subcore_barrier()

    @pl.loop(0, NCHUNK // 5)
    def _(m):
        ds_ = []
        for k in range(5):
            ds_.append(
                pltpu.async_copy(
                    ones_v, deg_sh.at[cidx_all.at[m * 5 + k]], sems, add=True
                )
            )
        for d in ds_:
            d.wait()

    plsc.subcore_barrier()

    pltpu.sync_copy(deg_sh.at[pl.ds(sid * NPS, NPS)], tmp_v)
    oo = pl.multiple_of(cid * N_P + sid * NPS, 8)
    pltpu.sync_copy(tmp_v, out_hbm.at[pl.ds(oo, NPS)])


@functools.partial(
    pl.kernel,
    out_type=jax.ShapeDtypeStruct((NC, N_P, C), jnp.float32),
    mesh=_mesh,
    scratch_types=[
        pltpu.VMEM((EPW,), jnp.int32),
        pltpu.VMEM((NCHUNK, CHUNK), jnp.int32),
        pltpu.VMEM((CHUNK, C), jnp.float32),
        pltpu.VMEM((CHUNK, C), jnp.float32),
        pltpu.VMEM_SHARED((N_P, C), jnp.float32),
        pltpu.SemaphoreType.DMA,
        pltpu.SemaphoreType.DMA,
        pltpu.SemaphoreType.DMA,
    ],
)
def _agg_kernel(
    g_hbm, row_hbm, col3d_hbm, out_hbm,
    ridx_all, cidx_all, rows_a, rows_b, acc_sh, sem_a, sem_b, semi,
):
    cid = lax.axis_index("c")
    sid = lax.axis_index("s")
    wid = sid * NC + cid

    roff = pl.multiple_of(wid * EPW, 8)
    rload = pltpu.async_copy(row_hbm.at[pl.ds(roff, EPW)], ridx_all, semi)
    cload = pltpu.async_copy(col3d_hbm.at[wid], cidx_all, semi)

    zval = jnp.zeros((16,), jnp.float32)

    @pl.loop(0, CHUNK)
    def _(r):
        for c16 in range(C // 16):
            rows_a[r, pl.ds(c16 * 16, 16)] = zval

    @pl.loop(sid * RPS, (sid + 1) * RPS)
    def _(j):
        pltpu.sync_copy(rows_a, acc_sh.at[pl.ds(j * CHUNK, CHUNK), :])

    rload.wait()
    cload.wait()
    plsc.subcore_barrier()

    pltpu.async_copy(g_hbm.at[ridx_all.at[pl.ds(0, CHUNK)]], rows_a, sem_a)
    pltpu.async_copy(g_hbm.at[ridx_all.at[pl.ds(CHUNK, CHUNK)]], rows_b, sem_b)

    @pl.loop(0, (NCHUNK - 1) // 2)
    def _(p):
        j = 2 * p
        pltpu.make_async_copy(g_hbm.at[ridx_all.at[pl.ds(j * CHUNK, CHUNK)]], rows_a, sem_a).wait()
        pltpu.sync_copy(rows_a, acc_sh.at[cidx_all.at[j]], add=True)
        pltpu.async_copy(g_hbm.at[ridx_all.at[pl.ds((j + 2) * CHUNK, CHUNK)]], rows_a, sem_a)

        pltpu.make_async_copy(g_hbm.at[ridx_all.at[pl.ds((j + 1) * CHUNK, CHUNK)]], rows_b, sem_b).wait()
        pltpu.sync_copy(rows_b, acc_sh.at[cidx_all.at[j + 1]], add=True)

        @pl.when(j + 3 < NCHUNK)
        def _():
            pltpu.async_copy(g_hbm.at[ridx_all.at[pl.ds((j + 3) * CHUNK, CHUNK)]], rows_b, sem_b)

    last = NCHUNK - 1
    pltpu.make_async_copy(g_hbm.at[ridx_all.at[pl.ds(last * CHUNK, CHUNK)]], rows_a, sem_a).wait()
    pltpu.sync_copy(rows_a, acc_sh.at[cidx_all.at[last]], add=True)

    plsc.subcore_barrier()

    o = pl.multiple_of(sid * (N_P // NS), 8)
    pltpu.sync_copy(acc_sh.at[pl.ds(o, N_P // NS), :],
                    out_hbm.at[cid, pl.ds(o, N_P // NS), :])


B2 = 1024
GRID2 = N_P // B2
B4 = 256
GRID4 = N_P // B4


def _k2_body(deg0_ref, deg1_ref, x_ref, w_ref, g_ref):
    s = jnp.sqrt(deg0_ref[...] + deg1_ref[...])
    h = lax.dot_general(
        x_ref[...], w_ref[...], (((1,), (1,)), ((), ())),
        preferred_element_type=jnp.float32,
    )
    g_ref[...] = h * s[:, None]


def _k4_body(s_part_ref, deg0_ref, deg1_ref, lb_ref, fw_ref, fb_ref, out_ref):
    sp = s_part_ref[...]
    st = sp[0] + sp[1]
    s = jnp.sqrt(deg0_ref[...] + deg1_ref[...])
    a = st * s[:, None] + lb_ref[...][None, :]
    out_ref[...] = (
        lax.dot_general(
            a, fw_ref[...], (((1,), (1,)), ((), ())),
            preferred_element_type=jnp.float32,
        )
        + fb_ref[...][None, :]
    )


_k2 = pl.pallas_call(
    _k2_body,
    out_shape=jax.ShapeDtypeStruct((N_P, C), jnp.float32),
    grid=(GRID2,),
    in_specs=[
        pl.BlockSpec((B2,), lambda i: (i,)),
        pl.BlockSpec((B2,), lambda i: (i + GRID2,)),
        pl.BlockSpec((B2, C), lambda i: (i, 0)),
        pl.BlockSpec((C, C), lambda i: (0, 0)),
    ],
    out_specs=pl.BlockSpec((B2, C), lambda i: (i, 0)),
)

_k4 = pl.pallas_call(
    _k4_body,
    out_shape=jax.ShapeDtypeStruct((N, C), jnp.float32),
    grid=(GRID4,),
    in_specs=[
        pl.BlockSpec((NC, B4, C), lambda i: (0, i, 0)),
        pl.BlockSpec((B4,), lambda i: (i,)),
        pl.BlockSpec((B4,), lambda i: (i + GRID4,)),
        pl.BlockSpec((C,), lambda i: (0,)),
        pl.BlockSpec((C, C), lambda i: (0, 0)),
        pl.BlockSpec((C,), lambda i: (0,)),
    ],
    out_specs=pl.BlockSpec((B4, C), lambda i: (i, 0)),
)


def kernel(x, edge_index, lin_w, lin_bias, fc_w, fc_b):
    col3d = edge_index[1].reshape(NW, NCHUNK, CHUNK)

    deg_part = _deg_kernel(col3d)
    g = _k2(deg_part, deg_part, x, lin_w)
    s_part = _agg_kernel(g, edge_index[0], col3d)
    return _k4(s_part, deg_part, deg_part, lin_bias, fc_w, fc_b)


# speedup vs baseline: 1.0932x; 1.0932x over previous
import functools

import jax
import jax.numpy as jnp
from jax import lax
from jax.experimental import pallas as pl
from jax.experimental.pallas import tpu as pltpu
from jax.experimental.pallas import tpu_sc as plsc

N = 10000
E = 320000
C = 128
N_P = 10240

NC = 2
NS = 16
NW = NC * NS
EPW = E // NW
CHUNK = 80
NCHUNK = EPW // CHUNK
RCHUNK = N_P // CHUNK
RPS = RCHUNK // NS

_mesh = plsc.VectorSubcoreMesh(
    core_axis_name="c", subcore_axis_name="s", num_cores=NC, num_subcores=NS
)


def _fill_vec16(ref, nwords, value):
    val = jnp.full((16,), value, dtype=jnp.float32)

    @pl.loop(0, nwords // 16)
    def _(i):
        ref[pl.ds(i * 16, 16)] = val


NPS = N_P // NS


@functools.partial(
    pl.kernel,
    out_type=jax.ShapeDtypeStruct((NC * N_P,), jnp.float32),
    mesh=_mesh,
    scratch_types=[
        pltpu.VMEM((NCHUNK, CHUNK), jnp.int32),
        pltpu.VMEM((CHUNK,), jnp.float32),
        pltpu.VMEM((NPS,), jnp.float32),
        pltpu.VMEM_SHARED((N_P,), jnp.float32),
        pltpu.SemaphoreType.DMA,
        pltpu.SemaphoreType.DMA,
    ],
)
def _deg_kernel(col3d_hbm, out_hbm, cidx_all, ones_v, tmp_v, deg_sh, semi, sems):
    cid = lax.axis_index("c")
    sid = lax.axis_index("s")
    wid = sid * NC + cid

    idx_load = pltpu.async_copy(col3d_hbm.at[wid], cidx_all, semi)

    _fill_vec16(ones_v, CHUNK, 1.0)
    _fill_vec16(tmp_v, NPS, 0.0)

    pltpu.sync_copy(tmp_v, deg_sh.at[pl.ds(sid * NPS, NPS)])
    idx_load.wait()
    plsc.subcore_barrier()

    @pl.loop(0, NCHUNK // 5)
    def _(m):
        ds_ = []
        for k in range(5):
            ds_.append(
                pltpu.async_copy(
                    ones_v, deg_sh.at[cidx_all.at[m * 5 + k]], sems, add=True
                )
            )
        for d in ds_:
            d.wait()

    plsc.subcore_barrier()

    pltpu.sync_copy(deg_sh.at[pl.ds(sid * NPS, NPS)], tmp_v)
    oo = pl.multiple_of(cid * N_P + sid * NPS, 8)
    pltpu.sync_copy(tmp_v, out_hbm.at[pl.ds(oo, NPS)])


@functools.partial(
    pl.kernel,
    out_type=jax.ShapeDtypeStruct((NC, N_P, C), jnp.float32),
    mesh=_mesh,
    scratch_types=[
        pltpu.VMEM((EPW,), jnp.int32),
        pltpu.VMEM((NCHUNK, CHUNK), jnp.int32),
        pltpu.VMEM((CHUNK, C), jnp.float32),
        pltpu.VMEM((CHUNK, C), jnp.float32),
        pltpu.VMEM_SHARED((N_P, C), jnp.float32),
        pltpu.SemaphoreType.DMA,
        pltpu.SemaphoreType.DMA,
        pltpu.SemaphoreType.DMA,
    ],
)
def _agg_kernel(
    g_hbm, row_hbm, col3d_hbm, out_hbm,
    ridx_all, cidx_all, rows_a, rows_b, acc_sh, sem_a, sem_b, semi,
):
    cid = lax.axis_index("c")
    sid = lax.axis_index("s")
    wid = sid * NC + cid

    roff = pl.multiple_of(wid * EPW, 8)
    rload = pltpu.async_copy(row_hbm.at[pl.ds(roff, EPW)], ridx_all, semi)
    cload = pltpu.async_copy(col3d_hbm.at[wid], cidx_all, semi)

    zval = jnp.zeros((16,), jnp.float32)

    @pl.loop(0, CHUNK)
    def _(r):
        for c16 in range(C // 16):
            rows_a[r, pl.ds(c16 * 16, 16)] = zval

    @pl.loop(sid * RPS, (sid + 1) * RPS)
    def _(j):
        pltpu.sync_copy(rows_a, acc_sh.at[pl.ds(j * CHUNK, CHUNK), :])

    rload.wait()
    cload.wait()
    plsc.subcore_barrier()

    pltpu.async_copy(g_hbm.at[ridx_all.at[pl.ds(0, CHUNK)]], rows_a, sem_a)
    pltpu.async_copy(g_hbm.at[ridx_all.at[pl.ds(CHUNK, CHUNK)]], rows_b, sem_b)

    @pl.loop(0, (NCHUNK - 1) // 2)
    def _(p):
        j = 2 * p
        pltpu.make_async_copy(g_hbm.at[ridx_all.at[pl.ds(j * CHUNK, CHUNK)]], rows_a, sem_a).wait()
        pltpu.sync_copy(rows_a, acc_sh.at[cidx_all.at[j]], add=True)
        pltpu.async_copy(g_hbm.at[ridx_all.at[pl.ds((j + 2) * CHUNK, CHUNK)]], rows_a, sem_a)

        pltpu.make_async_copy(g_hbm.at[ridx_all.at[pl.ds((j + 1) * CHUNK, CHUNK)]], rows_b, sem_b).wait()
        pltpu.sync_copy(rows_b, acc_sh.at[cidx_all.at[j + 1]], add=True)

        @pl.when(j + 3 < NCHUNK)
        def _():
            pltpu.async_copy(g_hbm.at[ridx_all.at[pl.ds((j + 3) * CHUNK, CHUNK)]], rows_b, sem_b)

    last = NCHUNK - 1
    pltpu.make_async_copy(g_hbm.at[ridx_all.at[pl.ds(last * CHUNK, CHUNK)]], rows_a, sem_a).wait()
    pltpu.sync_copy(rows_a, acc_sh.at[cidx_all.at[last]], add=True)

    plsc.subcore_barrier()

    o = pl.multiple_of(sid * (N_P // NS), 8)
    pltpu.sync_copy(acc_sh.at[pl.ds(o, N_P // NS), :],
                    out_hbm.at[cid, pl.ds(o, N_P // NS), :])


B2 = 1024
GRID2 = N_P // B2
B4 = 1024
GRID4 = N_P // B4


def _k2_body(deg0_ref, deg1_ref, x_ref, w_ref, g_ref):
    s = jnp.sqrt(deg0_ref[...] + deg1_ref[...])
    h = lax.dot_general(
        x_ref[...], w_ref[...], (((1,), (1,)), ((), ())),
        preferred_element_type=jnp.float32,
    )
    g_ref[...] = h * s[:, None]


def _k4_body(s_part_ref, deg0_ref, deg1_ref, lb_ref, fw_ref, fb_ref, out_ref):
    sp = s_part_ref[...]
    st = sp[0] + sp[1]
    s = jnp.sqrt(deg0_ref[...] + deg1_ref[...])
    a = st * s[:, None] + lb_ref[...][None, :]
    out_ref[...] = (
        lax.dot_general(
            a, fw_ref[...], (((1,), (1,)), ((), ())),
            preferred_element_type=jnp.float32,
        )
        + fb_ref[...][None, :]
    )


_k2 = pl.pallas_call(
    _k2_body,
    out_shape=jax.ShapeDtypeStruct((N_P, C), jnp.float32),
    grid=(GRID2,),
    in_specs=[
        pl.BlockSpec((B2,), lambda i: (i,)),
        pl.BlockSpec((B2,), lambda i: (i + GRID2,)),
        pl.BlockSpec((B2, C), lambda i: (i, 0)),
        pl.BlockSpec((C, C), lambda i: (0, 0)),
    ],
    out_specs=pl.BlockSpec((B2, C), lambda i: (i, 0)),
)

_k4 = pl.pallas_call(
    _k4_body,
    out_shape=jax.ShapeDtypeStruct((N, C), jnp.float32),
    grid=(GRID4,),
    in_specs=[
        pl.BlockSpec((NC, B4, C), lambda i: (0, i, 0)),
        pl.BlockSpec((B4,), lambda i: (i,)),
        pl.BlockSpec((B4,), lambda i: (i + GRID4,)),
        pl.BlockSpec((C,), lambda i: (0,)),
        pl.BlockSpec((C, C), lambda i: (0, 0)),
        pl.BlockSpec((C,), lambda i: (0,)),
    ],
    out_specs=pl.BlockSpec((B4, C), lambda i: (i, 0)),
)


def kernel(x, edge_index, lin_w, lin_bias, fc_w, fc_b):
    col3d = edge_index[1].reshape(NW, NCHUNK, CHUNK)

    deg_part = _deg_kernel(col3d)
    g = _k2(deg_part, deg_part, x, lin_w)
    s_part = _agg_kernel(g, edge_index[0], col3d)
    return _k4(s_part, deg_part, deg_part, lin_bias, fc_w, fc_b)


# speedup vs baseline: 1.2151x; 1.1115x over previous
import functools

import jax
import jax.numpy as jnp
from jax import lax
from jax.experimental import pallas as pl
from jax.experimental.pallas import tpu as pltpu
from jax.experimental.pallas import tpu_sc as plsc

N = 10000
E = 320000
C = 128
N_P = 10240

NC = 2
NS = 16
NW = NC * NS
EPW = E // NW
CHUNK = 80
NCHUNK = EPW // CHUNK
RCHUNK = N_P // CHUNK
RPS = RCHUNK // NS

_mesh = plsc.VectorSubcoreMesh(
    core_axis_name="c", subcore_axis_name="s", num_cores=NC, num_subcores=NS
)


def _fill_vec16(ref, nwords, value):
    val = jnp.full((16,), value, dtype=jnp.float32)

    @pl.loop(0, nwords // 16)
    def _(i):
        ref[pl.ds(i * 16, 16)] = val


NPS = N_P // NS


@functools.partial(
    pl.kernel,
    out_type=jax.ShapeDtypeStruct((NC * N_P,), jnp.float32),
    mesh=_mesh,
    scratch_types=[
        pltpu.VMEM((NCHUNK, CHUNK), jnp.int32),
        pltpu.VMEM((CHUNK,), jnp.float32),
        pltpu.VMEM((NPS,), jnp.float32),
        pltpu.VMEM_SHARED((N_P,), jnp.float32),
        pltpu.SemaphoreType.DMA,
        pltpu.SemaphoreType.DMA,
    ],
)
def _deg_kernel(col3d_hbm, out_hbm, cidx_all, ones_v, tmp_v, deg_sh, semi, sems):
    cid = lax.axis_index("c")
    sid = lax.axis_index("s")
    wid = sid * NC + cid

    idx_load = pltpu.async_copy(col3d_hbm.at[wid], cidx_all, semi)

    _fill_vec16(ones_v, CHUNK, 1.0)
    _fill_vec16(tmp_v, NPS, 0.0)

    pltpu.sync_copy(tmp_v, deg_sh.at[pl.ds(sid * NPS, NPS)])
    idx_load.wait()
    plsc.subcore_barrier()

    @pl.loop(0, NCHUNK // 5)
    def _(m):
        ds_ = []
        for k in range(5):
            ds_.append(
                pltpu.async_copy(
                    ones_v, deg_sh.at[cidx_all.at[m * 5 + k]], sems, add=True
                )
            )
        for d in ds_:
            d.wait()

    plsc.subcore_barrier()

    pltpu.sync_copy(deg_sh.at[pl.ds(sid * NPS, NPS)], tmp_v)
    oo = pl.multiple_of(cid * N_P + sid * NPS, 8)
    pltpu.sync_copy(tmp_v, out_hbm.at[pl.ds(oo, NPS)])


@functools.partial(
    pl.kernel,
    out_type=jax.ShapeDtypeStruct((NC, N_P, C), jnp.float32),
    mesh=_mesh,
    scratch_types=[
        pltpu.VMEM((EPW,), jnp.int32),
        pltpu.VMEM((CHUNK,), jnp.int32),
        pltpu.VMEM((CHUNK,), jnp.int32),
        pltpu.VMEM((CHUNK,), jnp.int32),
        pltpu.VMEM((CHUNK, C), jnp.float32),
        pltpu.VMEM((CHUNK, C), jnp.float32),
        pltpu.VMEM((CHUNK, C), jnp.float32),
        pltpu.VMEM_SHARED((N_P, C), jnp.float32),
        pltpu.SemaphoreType.DMA,
        pltpu.SemaphoreType.DMA,
        pltpu.SemaphoreType.DMA,
        pltpu.SemaphoreType.DMA,
        pltpu.SemaphoreType.DMA,
        pltpu.SemaphoreType.DMA,
        pltpu.SemaphoreType.DMA,
        pltpu.SemaphoreType.DMA,
        pltpu.SemaphoreType.DMA,
        pltpu.SemaphoreType.DMA,
    ],
)
def _agg_kernel(
    g_hbm, row_hbm, col_hbm, out_hbm,
    ridx_all, ci0, ci1, ci2, rw0, rw1, rw2, acc_sh,
    semi, sc0, sc1, sc2, sg0, sg1, sg2, ss0, ss1, ss2,
):
    cid = lax.axis_index("c")
    sid = lax.axis_index("s")
    wid = sid * NC + cid
    base = wid * EPW

    CI = (ci0, ci1, ci2)
    RW = (rw0, rw1, rw2)
    SC_ = (sc0, sc1, sc2)
    SG = (sg0, sg1, sg2)
    SS = (ss0, ss1, ss2)

    def fire_cidx(ch, b):
        co = pl.multiple_of(base + ch * CHUNK, 8)
        pltpu.async_copy(col_hbm.at[pl.ds(co, CHUNK)], CI[b], SC_[b])

    def wait_cidx(ch, b):
        co = pl.multiple_of(base + ch * CHUNK, 8)
        pltpu.make_async_copy(col_hbm.at[pl.ds(co, CHUNK)], CI[b], SC_[b]).wait()

    def fire_gather(ch, b):
        pltpu.async_copy(
            g_hbm.at[ridx_all.at[pl.ds(ch * CHUNK, CHUNK)]], RW[b], SG[b]
        )

    def wait_gather(ch, b):
        pltpu.make_async_copy(
            g_hbm.at[ridx_all.at[pl.ds(ch * CHUNK, CHUNK)]], RW[b], SG[b]
        ).wait()

    def fire_scatter(b):
        pltpu.async_copy(RW[b], acc_sh.at[CI[b]], SS[b], add=True)

    def wait_scatter(b):
        pltpu.make_async_copy(RW[b], acc_sh.at[CI[b]], SS[b]).wait()

    roff = pl.multiple_of(base, 8)
    rload = pltpu.async_copy(row_hbm.at[pl.ds(roff, EPW)], ridx_all, semi)
    for b in range(3):
        fire_cidx(b, b)

    zval = jnp.zeros((16,), jnp.float32)

    @pl.loop(0, CHUNK)
    def _(r):
        for c16 in range(C // 16):
            rw0[r, pl.ds(c16 * 16, 16)] = zval

    @pl.loop(sid * RPS, (sid + 1) * RPS)
    def _(j):
        pltpu.sync_copy(rw0, acc_sh.at[pl.ds(j * CHUNK, CHUNK), :])

    rload.wait()
    for b in range(3):
        fire_gather(b, b)
    plsc.subcore_barrier()

    @pl.loop(0, (NCHUNK - 2) // 3)
    def _(t):
        for k in range(3):
            i = 3 * t + k
            nb = (k + 1) % 3

            @pl.when(i >= 2)
            def _():
                wait_scatter(nb)
                fire_cidx(i + 1, nb)
                fire_gather(i + 1, nb)

            wait_gather(i, k)
            wait_cidx(i, k)
            fire_scatter(k)

    i0 = NCHUNK - 2
    wait_scatter(1)
    fire_cidx(i0 + 1, 1)
    fire_gather(i0 + 1, 1)
    wait_gather(i0, 0)
    wait_cidx(i0, 0)
    fire_scatter(0)

    wait_gather(i0 + 1, 1)
    wait_cidx(i0 + 1, 1)
    fire_scatter(1)

    wait_scatter(2)
    wait_scatter(0)
    wait_scatter(1)

    plsc.subcore_barrier()

    o = pl.multiple_of(sid * (N_P // NS), 8)
    pltpu.sync_copy(acc_sh.at[pl.ds(o, N_P // NS), :],
                    out_hbm.at[cid, pl.ds(o, N_P // NS), :])


B2 = 1024
GRID2 = N_P // B2
B4 = 1024
GRID4 = N_P // B4


def _k2_body(deg0_ref, deg1_ref, x_ref, w_ref, g_ref):
    s = jnp.sqrt(deg0_ref[...] + deg1_ref[...])
    h = lax.dot_general(
        x_ref[...], w_ref[...], (((1,), (1,)), ((), ())),
        preferred_element_type=jnp.float32,
    )
    g_ref[...] = h * s[:, None]


def _k4_body(s_part_ref, deg0_ref, deg1_ref, lb_ref, fw_ref, fb_ref, out_ref):
    sp = s_part_ref[...]
    st = sp[0] + sp[1]
    s = jnp.sqrt(deg0_ref[...] + deg1_ref[...])
    a = st * s[:, None] + lb_ref[...][None, :]
    out_ref[...] = (
        lax.dot_general(
            a, fw_ref[...], (((1,), (1,)), ((), ())),
            preferred_element_type=jnp.float32,
        )
        + fb_ref[...][None, :]
    )


_k2 = pl.pallas_call(
    _k2_body,
    out_shape=jax.ShapeDtypeStruct((N_P, C), jnp.float32),
    grid=(GRID2,),
    in_specs=[
        pl.BlockSpec((B2,), lambda i: (i,)),
        pl.BlockSpec((B2,), lambda i: (i + GRID2,)),
        pl.BlockSpec((B2, C), lambda i: (i, 0)),
        pl.BlockSpec((C, C), lambda i: (0, 0)),
    ],
    out_specs=pl.BlockSpec((B2, C), lambda i: (i, 0)),
)

_k4 = pl.pallas_call(
    _k4_body,
    out_shape=jax.ShapeDtypeStruct((N, C), jnp.float32),
    grid=(GRID4,),
    in_specs=[
        pl.BlockSpec((NC, B4, C), lambda i: (0, i, 0)),
        pl.BlockSpec((B4,), lambda i: (i,)),
        pl.BlockSpec((B4,), lambda i: (i + GRID4,)),
        pl.BlockSpec((C,), lambda i: (0,)),
        pl.BlockSpec((C, C), lambda i: (0, 0)),
        pl.BlockSpec((C,), lambda i: (0,)),
    ],
    out_specs=pl.BlockSpec((B4, C), lambda i: (i, 0)),
)


def kernel(x, edge_index, lin_w, lin_bias, fc_w, fc_b):
    col3d = edge_index[1].reshape(NW, NCHUNK, CHUNK)

    deg_part = _deg_kernel(col3d)
    g = _k2(deg_part, deg_part, x, lin_w)
    s_part = _agg_kernel(g, edge_index[0], edge_index[1])
    return _k4(s_part, deg_part, deg_part, lin_bias, fc_w, fc_b)


# speedup vs baseline: 1.2247x; 1.0079x over previous
import functools

import jax
import jax.numpy as jnp
from jax import lax
from jax.experimental import pallas as pl
from jax.experimental.pallas import tpu as pltpu
from jax.experimental.pallas import tpu_sc as plsc

N = 10000
E = 320000
C = 128
N_P = 10240

NC = 2
NS = 16
NW = NC * NS
EPW = E // NW
CHUNK = 80
NCHUNK = EPW // CHUNK
RCHUNK = N_P // CHUNK
RPS = RCHUNK // NS

_mesh = plsc.VectorSubcoreMesh(
    core_axis_name="c", subcore_axis_name="s", num_cores=NC, num_subcores=NS
)


def _fill_vec16(ref, nwords, value):
    val = jnp.full((16,), value, dtype=jnp.float32)

    @pl.loop(0, nwords // 16)
    def _(i):
        ref[pl.ds(i * 16, 16)] = val


NPS = N_P // NS


@functools.partial(
    pl.kernel,
    out_type=jax.ShapeDtypeStruct((NC * N_P,), jnp.float32),
    mesh=_mesh,
    scratch_types=[
        pltpu.VMEM((EPW,), jnp.int32),
        pltpu.VMEM((CHUNK,), jnp.float32),
        pltpu.VMEM((NPS,), jnp.float32),
        pltpu.VMEM_SHARED((N_P,), jnp.float32),
        pltpu.SemaphoreType.DMA,
        pltpu.SemaphoreType.DMA,
    ],
)
def _deg_kernel(col_hbm, out_hbm, cidx_all, ones_v, tmp_v, deg_sh, semi, sems):
    cid = lax.axis_index("c")
    sid = lax.axis_index("s")
    wid = sid * NC + cid

    coff = pl.multiple_of(wid * EPW, 8)
    idx_load = pltpu.async_copy(col_hbm.at[pl.ds(coff, EPW)], cidx_all, semi)

    _fill_vec16(ones_v, CHUNK, 1.0)
    _fill_vec16(tmp_v, NPS, 0.0)

    pltpu.sync_copy(tmp_v, deg_sh.at[pl.ds(sid * NPS, NPS)])
    idx_load.wait()
    plsc.subcore_barrier()

    @pl.loop(0, NCHUNK // 5)
    def _(m):
        ds_ = []
        for k in range(5):
            ix = cidx_all.at[pl.ds((m * 5 + k) * CHUNK, CHUNK)]
            ds_.append(pltpu.async_copy(ones_v, deg_sh.at[ix], sems, add=True))
        for d in ds_:
            d.wait()

    plsc.subcore_barrier()

    pltpu.sync_copy(deg_sh.at[pl.ds(sid * NPS, NPS)], tmp_v)
    oo = pl.multiple_of(cid * N_P + sid * NPS, 8)
    pltpu.sync_copy(tmp_v, out_hbm.at[pl.ds(oo, NPS)])


@functools.partial(
    pl.kernel,
    out_type=jax.ShapeDtypeStruct((NC, N_P, C), jnp.float32),
    mesh=_mesh,
    scratch_types=[
        pltpu.VMEM((EPW,), jnp.int32),
        pltpu.VMEM((CHUNK,), jnp.int32),
        pltpu.VMEM((CHUNK,), jnp.int32),
        pltpu.VMEM((CHUNK,), jnp.int32),
        pltpu.VMEM((CHUNK, C), jnp.float32),
        pltpu.VMEM((CHUNK, C), jnp.float32),
        pltpu.VMEM((CHUNK, C), jnp.float32),
        pltpu.VMEM_SHARED((N_P, C), jnp.float32),
        pltpu.SemaphoreType.DMA,
        pltpu.SemaphoreType.DMA,
        pltpu.SemaphoreType.DMA,
        pltpu.SemaphoreType.DMA,
        pltpu.SemaphoreType.DMA,
        pltpu.SemaphoreType.DMA,
        pltpu.SemaphoreType.DMA,
        pltpu.SemaphoreType.DMA,
        pltpu.SemaphoreType.DMA,
        pltpu.SemaphoreType.DMA,
    ],
)
def _agg_kernel(
    g_hbm, row_hbm, col_hbm, out_hbm,
    ridx_all, ci0, ci1, ci2, rw0, rw1, rw2, acc_sh,
    semi, sc0, sc1, sc2, sg0, sg1, sg2, ss0, ss1, ss2,
):
    cid = lax.axis_index("c")
    sid = lax.axis_index("s")
    wid = sid * NC + cid
    base = wid * EPW

    CI = (ci0, ci1, ci2)
    RW = (rw0, rw1, rw2)
    SC_ = (sc0, sc1, sc2)
    SG = (sg0, sg1, sg2)
    SS = (ss0, ss1, ss2)

    def fire_cidx(ch, b):
        co = pl.multiple_of(base + ch * CHUNK, 8)
        pltpu.async_copy(col_hbm.at[pl.ds(co, CHUNK)], CI[b], SC_[b])

    def wait_cidx(ch, b):
        co = pl.multiple_of(base + ch * CHUNK, 8)
        pltpu.make_async_copy(col_hbm.at[pl.ds(co, CHUNK)], CI[b], SC_[b]).wait()

    def fire_gather(ch, b):
        pltpu.async_copy(
            g_hbm.at[ridx_all.at[pl.ds(ch * CHUNK, CHUNK)]], RW[b], SG[b]
        )

    def wait_gather(ch, b):
        pltpu.make_async_copy(
            g_hbm.at[ridx_all.at[pl.ds(ch * CHUNK, CHUNK)]], RW[b], SG[b]
        ).wait()

    def fire_scatter(b):
        pltpu.async_copy(RW[b], acc_sh.at[CI[b]], SS[b], add=True)

    def wait_scatter(b):
        pltpu.make_async_copy(RW[b], acc_sh.at[CI[b]], SS[b]).wait()

    roff = pl.multiple_of(base, 8)
    rload = pltpu.async_copy(row_hbm.at[pl.ds(roff, EPW)], ridx_all, semi)
    for b in range(3):
        fire_cidx(b, b)

    zval = jnp.zeros((16,), jnp.float32)

    @pl.loop(0, CHUNK)
    def _(r):
        for c16 in range(C // 16):
            rw0[r, pl.ds(c16 * 16, 16)] = zval

    @pl.loop(sid * RPS, (sid + 1) * RPS)
    def _(j):
        pltpu.sync_copy(rw0, acc_sh.at[pl.ds(j * CHUNK, CHUNK), :])

    rload.wait()
    for b in range(3):
        fire_gather(b, b)
    plsc.subcore_barrier()

    @pl.loop(0, (NCHUNK - 2) // 3)
    def _(t):
        for k in range(3):
            i = 3 * t + k
            nb = (k + 1) % 3

            @pl.when(i >= 2)
            def _():
                wait_scatter(nb)
                fire_cidx(i + 1, nb)
                fire_gather(i + 1, nb)

            wait_gather(i, k)
            wait_cidx(i, k)
            fire_scatter(k)

    i0 = NCHUNK - 2
    wait_scatter(1)
    fire_cidx(i0 + 1, 1)
    fire_gather(i0 + 1, 1)
    wait_gather(i0, 0)
    wait_cidx(i0, 0)
    fire_scatter(0)

    wait_gather(i0 + 1, 1)
    wait_cidx(i0 + 1, 1)
    fire_scatter(1)

    wait_scatter(2)
    wait_scatter(0)
    wait_scatter(1)

    plsc.subcore_barrier()

    o = pl.multiple_of(sid * (N_P // NS), 8)
    pltpu.sync_copy(acc_sh.at[pl.ds(o, N_P // NS), :],
                    out_hbm.at[cid, pl.ds(o, N_P // NS), :])


B2 = 1024
GRID2 = N_P // B2
B4 = 1024
GRID4 = N_P // B4


def _k2_body(deg0_ref, deg1_ref, x_ref, w_ref, g_ref):
    s = jnp.sqrt(deg0_ref[...] + deg1_ref[...])
    h = lax.dot_general(
        x_ref[...], w_ref[...], (((1,), (1,)), ((), ())),
        preferred_element_type=jnp.float32,
    )
    g_ref[...] = h * s[:, None]


def _k4_body(s_part_ref, deg0_ref, deg1_ref, lb_ref, fw_ref, fb_ref, out_ref):
    sp = s_part_ref[...]
    st = sp[0] + sp[1]
    s = jnp.sqrt(deg0_ref[...] + deg1_ref[...])
    a = st * s[:, None] + lb_ref[...][None, :]
    out_ref[...] = (
        lax.dot_general(
            a, fw_ref[...], (((1,), (1,)), ((), ())),
            preferred_element_type=jnp.float32,
        )
        + fb_ref[...][None, :]
    )


_k2 = pl.pallas_call(
    _k2_body,
    out_shape=jax.ShapeDtypeStruct((N_P, C), jnp.float32),
    grid=(GRID2,),
    in_specs=[
        pl.BlockSpec((B2,), lambda i: (i,)),
        pl.BlockSpec((B2,), lambda i: (i + GRID2,)),
        pl.BlockSpec((B2, C), lambda i: (i, 0)),
        pl.BlockSpec((C, C), lambda i: (0, 0)),
    ],
    out_specs=pl.BlockSpec((B2, C), lambda i: (i, 0)),
)

_k4 = pl.pallas_call(
    _k4_body,
    out_shape=jax.ShapeDtypeStruct((N, C), jnp.float32),
    grid=(GRID4,),
    in_specs=[
        pl.BlockSpec((NC, B4, C), lambda i: (0, i, 0)),
        pl.BlockSpec((B4,), lambda i: (i,)),
        pl.BlockSpec((B4,), lambda i: (i + GRID4,)),
        pl.BlockSpec((C,), lambda i: (0,)),
        pl.BlockSpec((C, C), lambda i: (0, 0)),
        pl.BlockSpec((C,), lambda i: (0,)),
    ],
    out_specs=pl.BlockSpec((B4, C), lambda i: (i, 0)),
)


def kernel(x, edge_index, lin_w, lin_bias, fc_w, fc_b):
    col = edge_index[1]

    deg_part = _deg_kernel(col)
    g = _k2(deg_part, deg_part, x, lin_w)
    s_part = _agg_kernel(g, edge_index[0], col)
    return _k4(s_part, deg_part, deg_part, lin_bias, fc_w, fc_b)
